# Initial kernel scaffold; baseline (speedup 1.0000x reference)
import functools

import jax
import jax.numpy as jnp
from jax import lax
from jax.experimental import pallas as pl
from jax.experimental.pallas import tpu as pltpu
from jax.experimental.pallas import tpu_sc as plsc

NN = 10000
NE = 160000
D = 256
DE = 16
G = 64
DH = D // 2

NC = 2
NS = 16
LANES = 16

CH = 80
EPT = NE // NS
RPT = NN // NS
ZR = 125

BE = 2000
BN = 2000


def _edge_embed_body(ea, w1l, w1h, w2l, w2h, b1l, b1h, b2l, b2h,
                     o1l, o1h, o2l, o2h):
    a = ea[...]
    o1l[...] = jnp.dot(a, w1l[...], preferred_element_type=jnp.float32) + b1l[...]
    o1h[...] = jnp.dot(a, w1h[...], preferred_element_type=jnp.float32) + b1h[...]
    o2l[...] = jnp.dot(a, w2l[...], preferred_element_type=jnp.float32) + b2l[...]
    o2h[...] = jnp.dot(a, w2h[...], preferred_element_type=jnp.float32) + b2h[...]


def _edge_embed(edge_attr, We1, be1, We2, be2):
    full = lambda shape: pl.BlockSpec(shape, lambda i: (0, 0))
    out = jax.ShapeDtypeStruct((NE, DH), jnp.float32)
    return pl.pallas_call(
        _edge_embed_body,
        grid=(NE // BE,),
        in_specs=[
            pl.BlockSpec((BE, DE), lambda i: (i, 0)),
            full((DE, DH)), full((DE, DH)), full((DE, DH)), full((DE, DH)),
            full((1, DH)), full((1, DH)), full((1, DH)), full((1, DH)),
        ],
        out_specs=[pl.BlockSpec((BE, DH), lambda i: (i, 0))] * 4,
        out_shape=[out, out, out, out],
    )(edge_attr,
      We1[:, :DH], We1[:, DH:], We2[:, :DH], We2[:, DH:],
      be1[:DH].reshape(1, DH), be1[DH:].reshape(1, DH),
      be2[:DH].reshape(1, DH), be2[DH:].reshape(1, DH))


def _sc_body(xl_hbm, xh_hbm, el_hbm, eh_hbm, src_hbm, dst_hbm,
             ol_hbm, oh_hbm,
             src_v, dst_v, ev, xv, zbuf, agg_sh, sem):
    c = lax.axis_index("c")
    s = lax.axis_index("s")

    def zero_row(r, carry):
        for k in range(DH // LANES):
            zbuf[r, pl.ds(k * LANES, LANES)] = jnp.zeros((LANES,), jnp.float32)
        return carry
    lax.fori_loop(0, ZR, zero_row, 0)
    for q in range(RPT // ZR):
        pltpu.sync_copy(zbuf, agg_sh.at[pl.ds(s * RPT + q * ZR, ZR)])
    plsc.subcore_barrier()

    def chunk(j, carry):
        base = s * EPT + j * CH
        pltpu.sync_copy(src_hbm.at[pl.ds(base, CH)], src_v)
        pltpu.sync_copy(dst_hbm.at[pl.ds(base, CH)], dst_v)

        @pl.when(c == 0)
        def _():
            pltpu.sync_copy(el_hbm.at[pl.ds(base, CH)], ev)
            pltpu.async_copy(xl_hbm.at[src_v], xv, sem).wait()

        @pl.when(c == 1)
        def _():
            pltpu.sync_copy(eh_hbm.at[pl.ds(base, CH)], ev)
            pltpu.async_copy(xh_hbm.at[src_v], xv, sem).wait()

        def row(r, rc):
            for k in range(DH // LANES):
                sl = pl.ds(k * LANES, LANES)
                xv[r, sl] = jnp.maximum(xv[r, sl] + ev[r, sl], 0.0)
            return rc
        lax.fori_loop(0, CH, row, 0)

        pltpu.sync_copy(xv, agg_sh.at[dst_v], add=True)
        return carry
    lax.fori_loop(0, EPT // CH, chunk, 0)
    plsc.subcore_barrier()

    for q in range(RPT // ZR):
        rows = pl.ds(s * RPT + q * ZR, ZR)
        pltpu.sync_copy(agg_sh.at[rows], zbuf)

        @pl.when(c == 0)
        def _():
            pltpu.sync_copy(zbuf, ol_hbm.at[rows])

        @pl.when(c == 1)
        def _():
            pltpu.sync_copy(zbuf, oh_hbm.at[rows])


def _sc_agg(xl, xh, el, eh, src, dst):
    out = jax.ShapeDtypeStruct((NN, DH), jnp.float32)
    kern = pl.kernel(
        _sc_body,
        out_type=(out, out),
        mesh=plsc.VectorSubcoreMesh(core_axis_name="c", subcore_axis_name="s"),
        scratch_types=[
            pltpu.VMEM((CH,), jnp.int32),
            pltpu.VMEM((CH,), jnp.int32),
            pltpu.VMEM((CH, DH), jnp.float32),
            pltpu.VMEM((CH, DH), jnp.float32),
            pltpu.VMEM((ZR, DH), jnp.float32),
            pltpu.VMEM_SHARED((NN, DH), jnp.float32),
            pltpu.SemaphoreType.DMA,
        ],
    )
    return kern(xl, xh, el, eh, src, dst)


def _mlp1_body(x, al, ah, wa, wb, wp, ba, bb, bp, gs, bt, oh, ohl, ohh):
    xb = x[...]
    hin = xb + jnp.concatenate([al[...], ah[...]], axis=1)
    t = jnp.maximum(jnp.dot(hin, wa[...], preferred_element_type=jnp.float32)
                    + ba[...], 0.0)
    u = jnp.dot(t, wb[...], preferred_element_type=jnp.float32) + bb[...]
    v = jnp.maximum(u, 0.0) * gs[...] + bt[...]
    h = v + jnp.dot(xb, wp[...], preferred_element_type=jnp.float32) + bp[...]
    oh[...] = h
    ohl[...] = h[:, :DH]
    ohh[...] = h[:, DH:]


def _mlp1(x, al, ah, Wa, ba, Wb, bb, Wp, bp, gs, bt):
    fullw = lambda: pl.BlockSpec((D, D), lambda i: (0, 0))
    fullb = lambda: pl.BlockSpec((1, D), lambda i: (0, 0))
    return pl.pallas_call(
        _mlp1_body,
        grid=(NN // BN,),
        in_specs=[
            pl.BlockSpec((BN, D), lambda i: (i, 0)),
            pl.BlockSpec((BN, DH), lambda i: (i, 0)),
            pl.BlockSpec((BN, DH), lambda i: (i, 0)),
            fullw(), fullw(), fullw(),
            fullb(), fullb(), fullb(), fullb(), fullb(),
        ],
        out_specs=[
            pl.BlockSpec((BN, D), lambda i: (i, 0)),
            pl.BlockSpec((BN, DH), lambda i: (i, 0)),
            pl.BlockSpec((BN, DH), lambda i: (i, 0)),
        ],
        out_shape=[
            jax.ShapeDtypeStruct((NN, D), jnp.float32),
            jax.ShapeDtypeStruct((NN, DH), jnp.float32),
            jax.ShapeDtypeStruct((NN, DH), jnp.float32),
        ],
    )(x, al, ah, Wa, Wb, Wp,
      ba.reshape(1, D), bb.reshape(1, D), bp.reshape(1, D),
      gs.reshape(1, D), bt.reshape(1, D))


def _mlp2_pool_body(h, al, ah, wa, wb, wp, ba, bb, bp, gs, bt, bat, out):
    hb = h[...]
    hin = hb + jnp.concatenate([al[...], ah[...]], axis=1)
    t = jnp.maximum(jnp.dot(hin, wa[...], preferred_element_type=jnp.float32)
                    + ba[...], 0.0)
    u = jnp.dot(t, wb[...], preferred_element_type=jnp.float32) + bb[...]
    v = jnp.maximum(u, 0.0) * gs[...] + bt[...]
    h2 = v + jnp.dot(hb, wp[...], preferred_element_type=jnp.float32) + bp[...]

    ids = bat[0, 0, :]
    onehot = (ids[None, :] ==
              lax.broadcasted_iota(jnp.int32, (G, BN), 0)).astype(jnp.float32)

    @pl.when(pl.program_id(0) == 0)
    def _():
        out[...] = jnp.zeros_like(out)
    out[...] += jnp.dot(onehot, h2, preferred_element_type=jnp.float32)


def _mlp2_pool(h, al, ah, Wa, ba, Wb, bb, Wp, bp, gs, bt, batch):
    fullw = lambda: pl.BlockSpec((D, D), lambda i: (0, 0))
    fullb = lambda: pl.BlockSpec((1, D), lambda i: (0, 0))
    nb = NN // BN
    return pl.pallas_call(
        _mlp2_pool_body,
        grid=(nb,),
        in_specs=[
            pl.BlockSpec((BN, D), lambda i: (i, 0)),
            pl.BlockSpec((BN, DH), lambda i: (i, 0)),
            pl.BlockSpec((BN, DH), lambda i: (i, 0)),
            fullw(), fullw(), fullw(),
            fullb(), fullb(), fullb(), fullb(), fullb(),
            pl.BlockSpec((1, 1, BN), lambda i: (i, 0, 0)),
        ],
        out_specs=pl.BlockSpec((G, D), lambda i: (0, 0)),
        out_shape=jax.ShapeDtypeStruct((G, D), jnp.float32),
    )(h, al, ah, Wa, Wb, Wp,
      ba.reshape(1, D), bb.reshape(1, D), bp.reshape(1, D),
      gs.reshape(1, D), bt.reshape(1, D),
      batch.reshape(nb, 1, BN))


def kernel(x, edge_index, edge_attr, batch,
           W1a, b1a, W1b, b1b, We1, be1,
           W2a, b2a, W2b, b2b, We2, be2,
           Wp1, bp1, Wp2, bp2, g1, beta1, g2, beta2):
    src = edge_index[0]
    dst = edge_index[1]
    bn_scale = 1.0 / jnp.sqrt(jnp.float32(1.0 + 1e-5))
    g1s = g1 * bn_scale
    g2s = g2 * bn_scale

    e1l, e1h, e2l, e2h = _edge_embed(edge_attr, We1, be1, We2, be2)

    xl = x[:, :DH]
    xh = x[:, DH:]
    a1l, a1h = _sc_agg(xl, xh, e1l, e1h, src, dst)
    h, hl, hh = _mlp1(x, a1l, a1h, W1a, b1a, W1b, b1b, Wp1, bp1, g1s, beta1)
    a2l, a2h = _sc_agg(hl, hh, e2l, e2h, src, dst)
    out = _mlp2_pool(h, a2l, a2h, W2a, b2a, W2b, b2b, Wp2, bp2, g2s, beta2,
                     batch)
    return out


# speedup vs baseline: 1.9787x; 1.9787x over previous
import functools

import jax
import jax.numpy as jnp
from jax import lax
from jax.experimental import pallas as pl
from jax.experimental.pallas import tpu as pltpu
from jax.experimental.pallas import tpu_sc as plsc

NN = 10000
NE = 160000
D = 256
DE = 16
G = 64
DH = D // 2

NC = 2
NS = 16
LANES = 16

CH = 80
EPT = NE // NS
RPT = 624
ZR = 208
NTAIL = NN - NS * RPT

BE = 2000
BN = 2000


def _edge_embed_body(ea, w1l, w1h, w2l, w2h, b1l, b1h, b2l, b2h,
                     o1l, o1h, o2l, o2h):
    a = ea[...]
    o1l[...] = jnp.dot(a, w1l[...], preferred_element_type=jnp.float32) + b1l[...]
    o1h[...] = jnp.dot(a, w1h[...], preferred_element_type=jnp.float32) + b1h[...]
    o2l[...] = jnp.dot(a, w2l[...], preferred_element_type=jnp.float32) + b2l[...]
    o2h[...] = jnp.dot(a, w2h[...], preferred_element_type=jnp.float32) + b2h[...]


def _edge_embed(edge_attr, We1, be1, We2, be2):
    full = lambda shape: pl.BlockSpec(shape, lambda i: (0, 0))
    out = jax.ShapeDtypeStruct((NE, DH), jnp.float32)
    return pl.pallas_call(
        _edge_embed_body,
        grid=(NE // BE,),
        in_specs=[
            pl.BlockSpec((BE, DE), lambda i: (i, 0)),
            full((DE, DH)), full((DE, DH)), full((DE, DH)), full((DE, DH)),
            full((1, DH)), full((1, DH)), full((1, DH)), full((1, DH)),
        ],
        out_specs=[pl.BlockSpec((BE, DH), lambda i: (i, 0))] * 4,
        out_shape=[out, out, out, out],
    )(edge_attr,
      We1[:, :DH], We1[:, DH:], We2[:, :DH], We2[:, DH:],
      be1[:DH].reshape(1, DH), be1[DH:].reshape(1, DH),
      be2[:DH].reshape(1, DH), be2[DH:].reshape(1, DH))


def _sc_body(xl_hbm, xh_hbm, el_hbm, eh_hbm, src_hbm, dst_hbm,
             ol_hbm, oh_hbm,
             src_v, dst_v, ev, xv, zbuf, agg_sh, sem):
    c = lax.axis_index("c")
    s = lax.axis_index("s")

    def zero_row(r, carry):
        for k in range(DH // LANES):
            zbuf[r, pl.ds(k * LANES, LANES)] = jnp.zeros((LANES,), jnp.float32)
        return carry
    lax.fori_loop(0, ZR, zero_row, 0)
    for q in range(RPT // ZR):
        pltpu.sync_copy(zbuf, agg_sh.at[pl.ds(s * RPT + q * ZR, ZR)])

    @pl.when(s == NS - 1)
    def _():
        pltpu.sync_copy(zbuf.at[pl.ds(0, NTAIL)],
                        agg_sh.at[pl.ds(NS * RPT, NTAIL)])
    plsc.subcore_barrier()

    def chunk(j, carry):
        base = s * EPT + j * CH
        pltpu.sync_copy(src_hbm.at[pl.ds(base, CH)], src_v)
        pltpu.sync_copy(dst_hbm.at[pl.ds(base, CH)], dst_v)

        @pl.when(c == 0)
        def _():
            pltpu.sync_copy(el_hbm.at[pl.ds(base, CH)], ev)
            pltpu.async_copy(xl_hbm.at[src_v], xv, sem).wait()

        @pl.when(c == 1)
        def _():
            pltpu.sync_copy(eh_hbm.at[pl.ds(base, CH)], ev)
            pltpu.async_copy(xh_hbm.at[src_v], xv, sem).wait()

        def row(r, rc):
            for k in range(DH // LANES):
                sl = pl.ds(k * LANES, LANES)
                xv[r, sl] = jnp.maximum(xv[r, sl] + ev[r, sl], 0.0)
            return rc
        lax.fori_loop(0, CH, row, 0)

        pltpu.sync_copy(xv, agg_sh.at[dst_v], add=True)
        return carry
    lax.fori_loop(0, EPT // CH, chunk, 0)
    plsc.subcore_barrier()

    for q in range(RPT // ZR):
        rows = pl.ds(s * RPT + q * ZR, ZR)
        pltpu.sync_copy(agg_sh.at[rows], zbuf)

        @pl.when(c == 0)
        def _():
            pltpu.sync_copy(zbuf, ol_hbm.at[rows])

        @pl.when(c == 1)
        def _():
            pltpu.sync_copy(zbuf, oh_hbm.at[rows])

    @pl.when(s == NS - 1)
    def _():
        tail = pl.ds(NS * RPT, NTAIL)
        pltpu.sync_copy(agg_sh.at[tail], zbuf.at[pl.ds(0, NTAIL)])

        @pl.when(c == 0)
        def _():
            pltpu.sync_copy(zbuf.at[pl.ds(0, NTAIL)], ol_hbm.at[tail])

        @pl.when(c == 1)
        def _():
            pltpu.sync_copy(zbuf.at[pl.ds(0, NTAIL)], oh_hbm.at[tail])


def _sc_agg(xl, xh, el, eh, src, dst):
    out = jax.ShapeDtypeStruct((NN, DH), jnp.float32)
    kern = pl.kernel(
        _sc_body,
        out_type=(out, out),
        mesh=plsc.VectorSubcoreMesh(core_axis_name="c", subcore_axis_name="s"),
        scratch_types=[
            pltpu.VMEM((CH,), jnp.int32),
            pltpu.VMEM((CH,), jnp.int32),
            pltpu.VMEM((CH, DH), jnp.float32),
            pltpu.VMEM((CH, DH), jnp.float32),
            pltpu.VMEM((ZR, DH), jnp.float32),
            pltpu.VMEM_SHARED((NN, DH), jnp.float32),
            pltpu.SemaphoreType.DMA,
        ],
    )
    return kern(xl, xh, el, eh, src, dst)


def _mlp1_body(x, al, ah, wa, wb, wp, ba, bb, bp, gs, bt, oh, ohl, ohh):
    xb = x[...]
    hin = xb + jnp.concatenate([al[...], ah[...]], axis=1)
    t = jnp.maximum(jnp.dot(hin, wa[...], preferred_element_type=jnp.float32)
                    + ba[...], 0.0)
    u = jnp.dot(t, wb[...], preferred_element_type=jnp.float32) + bb[...]
    v = jnp.maximum(u, 0.0) * gs[...] + bt[...]
    h = v + jnp.dot(xb, wp[...], preferred_element_type=jnp.float32) + bp[...]
    oh[...] = h
    ohl[...] = h[:, :DH]
    ohh[...] = h[:, DH:]


def _mlp1(x, al, ah, Wa, ba, Wb, bb, Wp, bp, gs, bt):
    fullw = lambda: pl.BlockSpec((D, D), lambda i: (0, 0))
    fullb = lambda: pl.BlockSpec((1, D), lambda i: (0, 0))
    return pl.pallas_call(
        _mlp1_body,
        grid=(NN // BN,),
        in_specs=[
            pl.BlockSpec((BN, D), lambda i: (i, 0)),
            pl.BlockSpec((BN, DH), lambda i: (i, 0)),
            pl.BlockSpec((BN, DH), lambda i: (i, 0)),
            fullw(), fullw(), fullw(),
            fullb(), fullb(), fullb(), fullb(), fullb(),
        ],
        out_specs=[
            pl.BlockSpec((BN, D), lambda i: (i, 0)),
            pl.BlockSpec((BN, DH), lambda i: (i, 0)),
            pl.BlockSpec((BN, DH), lambda i: (i, 0)),
        ],
        out_shape=[
            jax.ShapeDtypeStruct((NN, D), jnp.float32),
            jax.ShapeDtypeStruct((NN, DH), jnp.float32),
            jax.ShapeDtypeStruct((NN, DH), jnp.float32),
        ],
    )(x, al, ah, Wa, Wb, Wp,
      ba.reshape(1, D), bb.reshape(1, D), bp.reshape(1, D),
      gs.reshape(1, D), bt.reshape(1, D))


def _mlp2_pool_body(h, al, ah, wa, wb, wp, ba, bb, bp, gs, bt, bat, out):
    hb = h[...]
    hin = hb + jnp.concatenate([al[...], ah[...]], axis=1)
    t = jnp.maximum(jnp.dot(hin, wa[...], preferred_element_type=jnp.float32)
                    + ba[...], 0.0)
    u = jnp.dot(t, wb[...], preferred_element_type=jnp.float32) + bb[...]
    v = jnp.maximum(u, 0.0) * gs[...] + bt[...]
    h2 = v + jnp.dot(hb, wp[...], preferred_element_type=jnp.float32) + bp[...]

    ids = bat[0, 0, :]
    onehot = (ids[None, :] ==
              lax.broadcasted_iota(jnp.int32, (G, BN), 0)).astype(jnp.float32)

    @pl.when(pl.program_id(0) == 0)
    def _():
        out[...] = jnp.zeros_like(out)
    out[...] += jnp.dot(onehot, h2, preferred_element_type=jnp.float32)


def _mlp2_pool(h, al, ah, Wa, ba, Wb, bb, Wp, bp, gs, bt, batch):
    fullw = lambda: pl.BlockSpec((D, D), lambda i: (0, 0))
    fullb = lambda: pl.BlockSpec((1, D), lambda i: (0, 0))
    nb = NN // BN
    return pl.pallas_call(
        _mlp2_pool_body,
        grid=(nb,),
        in_specs=[
            pl.BlockSpec((BN, D), lambda i: (i, 0)),
            pl.BlockSpec((BN, DH), lambda i: (i, 0)),
            pl.BlockSpec((BN, DH), lambda i: (i, 0)),
            fullw(), fullw(), fullw(),
            fullb(), fullb(), fullb(), fullb(), fullb(),
            pl.BlockSpec((1, 1, BN), lambda i: (i, 0, 0)),
        ],
        out_specs=pl.BlockSpec((G, D), lambda i: (0, 0)),
        out_shape=jax.ShapeDtypeStruct((G, D), jnp.float32),
    )(h, al, ah, Wa, Wb, Wp,
      ba.reshape(1, D), bb.reshape(1, D), bp.reshape(1, D),
      gs.reshape(1, D), bt.reshape(1, D),
      batch.reshape(nb, 1, BN))


def kernel(x, edge_index, edge_attr, batch,
           W1a, b1a, W1b, b1b, We1, be1,
           W2a, b2a, W2b, b2b, We2, be2,
           Wp1, bp1, Wp2, bp2, g1, beta1, g2, beta2):
    src = edge_index[0]
    dst = edge_index[1]
    bn_scale = 1.0 / jnp.sqrt(jnp.float32(1.0 + 1e-5))
    g1s = g1 * bn_scale
    g2s = g2 * bn_scale

    e1l, e1h, e2l, e2h = _edge_embed(edge_attr, We1, be1, We2, be2)

    xl = x[:, :DH]
    xh = x[:, DH:]
    a1l, a1h = _sc_agg(xl, xh, e1l, e1h, src, dst)
    h, hl, hh = _mlp1(x, a1l, a1h, W1a, b1a, W1b, b1b, Wp1, bp1, g1s, beta1)
    a2l, a2h = _sc_agg(hl, hh, e2l, e2h, src, dst)
    out = _mlp2_pool(h, a2l, a2h, W2a, b2a, W2b, b2b, Wp2, bp2, g2s, beta2,
                     batch)
    return out


# speedup vs baseline: 3.9412x; 1.9918x over previous
import functools

import jax
import jax.numpy as jnp
from jax import lax
from jax.experimental import pallas as pl
from jax.experimental.pallas import tpu as pltpu
from jax.experimental.pallas import tpu_sc as plsc

NN = 10000
NE = 160000
D = 256
DE = 16
G = 64
DH = D // 2

NC = 2
NS = 16
LANES = 16

CH = 40
EPT = NE // NS
NCH = EPT // CH
RPT = 624
ZR = 48
NTAIL = NN - NS * RPT

BE = 2000
BN = 2000


def _edge_embed_body(ea, w1l, w1h, w2l, w2h, b1l, b1h, b2l, b2h,
                     o1l, o1h, o2l, o2h):
    a = ea[...]
    o1l[...] = jnp.dot(a, w1l[...], preferred_element_type=jnp.float32) + b1l[...]
    o1h[...] = jnp.dot(a, w1h[...], preferred_element_type=jnp.float32) + b1h[...]
    o2l[...] = jnp.dot(a, w2l[...], preferred_element_type=jnp.float32) + b2l[...]
    o2h[...] = jnp.dot(a, w2h[...], preferred_element_type=jnp.float32) + b2h[...]


def _edge_embed(edge_attr, We1, be1, We2, be2):
    full = lambda shape: pl.BlockSpec(shape, lambda i: (0, 0))
    out = jax.ShapeDtypeStruct((NE, DH), jnp.float32)
    return pl.pallas_call(
        _edge_embed_body,
        grid=(NE // BE,),
        in_specs=[
            pl.BlockSpec((BE, DE), lambda i: (i, 0)),
            full((DE, DH)), full((DE, DH)), full((DE, DH)), full((DE, DH)),
            full((1, DH)), full((1, DH)), full((1, DH)), full((1, DH)),
        ],
        out_specs=[pl.BlockSpec((BE, DH), lambda i: (i, 0))] * 4,
        out_shape=[out, out, out, out],
    )(edge_attr,
      We1[:, :DH], We1[:, DH:], We2[:, :DH], We2[:, DH:],
      be1[:DH].reshape(1, DH), be1[DH:].reshape(1, DH),
      be2[:DH].reshape(1, DH), be2[DH:].reshape(1, DH))


def _sc_body(xl_hbm, xh_hbm, el_hbm, eh_hbm, src_hbm, dst_hbm,
             ol_hbm, oh_hbm,
             srcm, dv0, dv1, dv2, dv3, ev0, ev1, xv0, xv1, xv2, xv3,
             zbuf, agg_sh, l0, l1, l2, l3, s0, s1, s2, s3):
    c = lax.axis_index("c")
    s = lax.axis_index("s")
    dst_v = (dv0, dv1, dv2, dv3)
    ev = (ev0, ev1)
    xv = (xv0, xv1, xv2, xv3)
    lsem = (l0, l1, l2, l3)
    ssem = (s0, s1, s2, s3)

    def zero_row(r, carry):
        for k in range(DH // LANES):
            zbuf[r, pl.ds(k * LANES, LANES)] = jnp.zeros((LANES,), jnp.float32)
        return carry
    lax.fori_loop(0, ZR, zero_row, 0)
    for q in range(RPT // ZR):
        pltpu.sync_copy(zbuf, agg_sh.at[pl.ds(s * RPT + q * ZR, ZR)])

    @pl.when(s == NS - 1)
    def _():
        pltpu.sync_copy(zbuf.at[pl.ds(0, NTAIL)],
                        agg_sh.at[pl.ds(NS * RPT, NTAIL)])
    plsc.subcore_barrier()

    pltpu.sync_copy(src_hbm.at[pl.ds(s * EPT, EPT)], srcm)

    def issue_loads(j, b, eb):
        base = s * EPT + j * CH
        pltpu.async_copy(dst_hbm.at[pl.ds(base, CH)], dst_v[b], lsem[b])
        idx = srcm.at[pl.ds(j * CH, CH)]

        @pl.when(c == 0)
        def _():
            pltpu.async_copy(el_hbm.at[pl.ds(base, CH)], ev[eb], lsem[b])
            pltpu.async_copy(xl_hbm.at[idx], xv[b], lsem[b])

        @pl.when(c == 1)
        def _():
            pltpu.async_copy(eh_hbm.at[pl.ds(base, CH)], ev[eb], lsem[b])
            pltpu.async_copy(xh_hbm.at[idx], xv[b], lsem[b])

    def drain_loads(j, b, eb):
        base = s * EPT + j * CH
        pltpu.make_async_copy(dst_hbm.at[pl.ds(base, CH)], dst_v[b],
                              lsem[b]).wait()
        pltpu.make_async_copy(el_hbm.at[pl.ds(base, CH)], ev[eb],
                              lsem[b]).wait()
        pltpu.make_async_copy(xl_hbm.at[srcm.at[pl.ds(j * CH, CH)]], xv[b],
                              lsem[b]).wait()

    def compute(b, eb):
        def row(r, rc):
            for k in range(DH // LANES):
                sl = pl.ds(k * LANES, LANES)
                xv[b][r, sl] = jnp.maximum(xv[b][r, sl] + ev[eb][r, sl], 0.0)
            return rc
        lax.fori_loop(0, CH, row, 0)

    def wait_scatter(b):
        pltpu.make_async_copy(xv[b], agg_sh.at[dst_v[b]], ssem[b]).wait()

    def slot(j, u, do_issue, guard_wait):
        drain_loads(j, u, u % 2)
        compute(u, u % 2)
        pltpu.async_copy(xv[u], agg_sh.at[dst_v[u]], ssem[u], add=True)
        if guard_wait:
            @pl.when(j >= 2)
            def _():
                wait_scatter((u + 2) % 4)
        else:
            wait_scatter((u + 2) % 4)
        if do_issue:
            issue_loads(j + 2, (u + 2) % 4, u % 2)

    issue_loads(0, 0, 0)
    issue_loads(1, 1, 1)

    def quad(g, carry):
        j = 4 * g
        for u in range(4):
            slot(j + u, u, do_issue=True, guard_wait=(u < 2))
        return carry
    lax.fori_loop(0, (NCH - 2) // 4, quad, 0)
    slot(NCH - 2, 0, do_issue=False, guard_wait=False)
    slot(NCH - 1, 1, do_issue=False, guard_wait=False)
    wait_scatter(0)
    wait_scatter(1)
    plsc.subcore_barrier()

    for q in range(3):
        rows = pl.ds(s * RPT + q * (RPT // 3), RPT // 3)

        @pl.when(c == 0)
        def _():
            pltpu.sync_copy(agg_sh.at[rows], ol_hbm.at[rows])

        @pl.when(c == 1)
        def _():
            pltpu.sync_copy(agg_sh.at[rows], oh_hbm.at[rows])

    @pl.when(s == NS - 1)
    def _():
        tail = pl.ds(NS * RPT, NTAIL)

        @pl.when(c == 0)
        def _():
            pltpu.sync_copy(agg_sh.at[tail], ol_hbm.at[tail])

        @pl.when(c == 1)
        def _():
            pltpu.sync_copy(agg_sh.at[tail], oh_hbm.at[tail])


def _sc_agg(xl, xh, el, eh, src, dst):
    out = jax.ShapeDtypeStruct((NN, DH), jnp.float32)
    kern = pl.kernel(
        _sc_body,
        out_type=(out, out),
        mesh=plsc.VectorSubcoreMesh(core_axis_name="c", subcore_axis_name="s"),
        scratch_types=(
            [pltpu.VMEM((EPT,), jnp.int32)]
            + [pltpu.VMEM((CH,), jnp.int32)] * 4
            + [pltpu.VMEM((CH, DH), jnp.float32)] * 6
            + [pltpu.VMEM((ZR, DH), jnp.float32),
               pltpu.VMEM_SHARED((NN, DH), jnp.float32)]
            + [pltpu.SemaphoreType.DMA] * 8
        ),
    )
    return kern(xl, xh, el, eh, src, dst)


def _mlp1_body(x, al, ah, wa, wb, wp, ba, bb, bp, gs, bt, oh, ohl, ohh):
    xb = x[...]
    hin = xb + jnp.concatenate([al[...], ah[...]], axis=1)
    t = jnp.maximum(jnp.dot(hin, wa[...], preferred_element_type=jnp.float32)
                    + ba[...], 0.0)
    u = jnp.dot(t, wb[...], preferred_element_type=jnp.float32) + bb[...]
    v = jnp.maximum(u, 0.0) * gs[...] + bt[...]
    h = v + jnp.dot(xb, wp[...], preferred_element_type=jnp.float32) + bp[...]
    oh[...] = h
    ohl[...] = h[:, :DH]
    ohh[...] = h[:, DH:]


def _mlp1(x, al, ah, Wa, ba, Wb, bb, Wp, bp, gs, bt):
    fullw = lambda: pl.BlockSpec((D, D), lambda i: (0, 0))
    fullb = lambda: pl.BlockSpec((1, D), lambda i: (0, 0))
    return pl.pallas_call(
        _mlp1_body,
        grid=(NN // BN,),
        in_specs=[
            pl.BlockSpec((BN, D), lambda i: (i, 0)),
            pl.BlockSpec((BN, DH), lambda i: (i, 0)),
            pl.BlockSpec((BN, DH), lambda i: (i, 0)),
            fullw(), fullw(), fullw(),
            fullb(), fullb(), fullb(), fullb(), fullb(),
        ],
        out_specs=[
            pl.BlockSpec((BN, D), lambda i: (i, 0)),
            pl.BlockSpec((BN, DH), lambda i: (i, 0)),
            pl.BlockSpec((BN, DH), lambda i: (i, 0)),
        ],
        out_shape=[
            jax.ShapeDtypeStruct((NN, D), jnp.float32),
            jax.ShapeDtypeStruct((NN, DH), jnp.float32),
            jax.ShapeDtypeStruct((NN, DH), jnp.float32),
        ],
    )(x, al, ah, Wa, Wb, Wp,
      ba.reshape(1, D), bb.reshape(1, D), bp.reshape(1, D),
      gs.reshape(1, D), bt.reshape(1, D))


def _mlp2_pool_body(h, al, ah, wa, wb, wp, ba, bb, bp, gs, bt, bat, out):
    hb = h[...]
    hin = hb + jnp.concatenate([al[...], ah[...]], axis=1)
    t = jnp.maximum(jnp.dot(hin, wa[...], preferred_element_type=jnp.float32)
                    + ba[...], 0.0)
    u = jnp.dot(t, wb[...], preferred_element_type=jnp.float32) + bb[...]
    v = jnp.maximum(u, 0.0) * gs[...] + bt[...]
    h2 = v + jnp.dot(hb, wp[...], preferred_element_type=jnp.float32) + bp[...]

    ids = bat[0, 0, :]
    onehot = (ids[None, :] ==
              lax.broadcasted_iota(jnp.int32, (G, BN), 0)).astype(jnp.float32)

    @pl.when(pl.program_id(0) == 0)
    def _():
        out[...] = jnp.zeros_like(out)
    out[...] += jnp.dot(onehot, h2, preferred_element_type=jnp.float32)


def _mlp2_pool(h, al, ah, Wa, ba, Wb, bb, Wp, bp, gs, bt, batch):
    fullw = lambda: pl.BlockSpec((D, D), lambda i: (0, 0))
    fullb = lambda: pl.BlockSpec((1, D), lambda i: (0, 0))
    nb = NN // BN
    return pl.pallas_call(
        _mlp2_pool_body,
        grid=(nb,),
        in_specs=[
            pl.BlockSpec((BN, D), lambda i: (i, 0)),
            pl.BlockSpec((BN, DH), lambda i: (i, 0)),
            pl.BlockSpec((BN, DH), lambda i: (i, 0)),
            fullw(), fullw(), fullw(),
            fullb(), fullb(), fullb(), fullb(), fullb(),
            pl.BlockSpec((1, 1, BN), lambda i: (i, 0, 0)),
        ],
        out_specs=pl.BlockSpec((G, D), lambda i: (0, 0)),
        out_shape=jax.ShapeDtypeStruct((G, D), jnp.float32),
    )(h, al, ah, Wa, Wb, Wp,
      ba.reshape(1, D), bb.reshape(1, D), bp.reshape(1, D),
      gs.reshape(1, D), bt.reshape(1, D),
      batch.reshape(nb, 1, BN))


def kernel(x, edge_index, edge_attr, batch,
           W1a, b1a, W1b, b1b, We1, be1,
           W2a, b2a, W2b, b2b, We2, be2,
           Wp1, bp1, Wp2, bp2, g1, beta1, g2, beta2):
    src = edge_index[0]
    dst = edge_index[1]
    bn_scale = 1.0 / jnp.sqrt(jnp.float32(1.0 + 1e-5))
    g1s = g1 * bn_scale
    g2s = g2 * bn_scale

    e1l, e1h, e2l, e2h = _edge_embed(edge_attr, We1, be1, We2, be2)

    xl = x[:, :DH]
    xh = x[:, DH:]
    a1l, a1h = _sc_agg(xl, xh, e1l, e1h, src, dst)
    h, hl, hh = _mlp1(x, a1l, a1h, W1a, b1a, W1b, b1b, Wp1, bp1, g1s, beta1)
    a2l, a2h = _sc_agg(hl, hh, e2l, e2h, src, dst)
    out = _mlp2_pool(h, a2l, a2h, W2a, b2a, W2b, b2b, Wp2, bp2, g2s, beta2,
                     batch)
    return out


# speedup vs baseline: 3.9839x; 1.0108x over previous
import functools

import jax
import jax.numpy as jnp
import numpy as np
from jax import lax
from jax.experimental import pallas as pl
from jax.experimental.pallas import tpu as pltpu
from jax.experimental.pallas import tpu_sc as plsc

NN = 10000
NE = 160000
D = 256
DE = 16
G = 64
DH = D // 2

NC = 2
NS = 16
LANES = 16

CH = 40
EPT = NE // NS
NCH = EPT // CH
RPT = 624
ZR = 48
NTAIL = NN - NS * RPT

BE = 2000
BN = 2000

_L = np.concatenate([np.arange(32 * g, 32 * g + 16) for g in range(DH // 32)])
_PLO = np.concatenate([_L, DH + _L])
_PHI = _PLO + 16


def _edge_embed_body(ea, w1l, w1h, w2l, w2h, b1l, b1h, b2l, b2h, o1, o2):
    a = ea[...].astype(jnp.bfloat16)

    def emb(w, b):
        v = jnp.dot(a, w[...], preferred_element_type=jnp.float32) + b[...]
        return lax.bitcast_convert_type(v.astype(jnp.bfloat16),
                                        jnp.uint16).astype(jnp.uint32)

    o1[...] = emb(w1l, b1l) | (emb(w1h, b1h) << 16)
    o2[...] = emb(w2l, b2l) | (emb(w2h, b2h) << 16)


def _edge_embed(edge_attr, We1, be1, We2, be2):
    full = lambda shape: pl.BlockSpec(shape, lambda i: (0, 0))
    out = jax.ShapeDtypeStruct((NE, D // 2), jnp.uint32)
    ws = []
    bs = []
    for We, be in ((We1, be1), (We2, be2)):
        ws += [We[:, _PLO].astype(jnp.bfloat16), We[:, _PHI].astype(jnp.bfloat16)]
        bs += [be[_PLO].reshape(1, D // 2), be[_PHI].reshape(1, D // 2)]
    return pl.pallas_call(
        _edge_embed_body,
        grid=(NE // BE,),
        in_specs=(
            [pl.BlockSpec((BE, DE), lambda i: (i, 0))]
            + [full((DE, D // 2))] * 4
            + [full((1, D // 2))] * 4
        ),
        out_specs=[pl.BlockSpec((BE, D // 2), lambda i: (i, 0))] * 2,
        out_shape=[out, out],
    )(edge_attr, *ws, *bs)


def _split_body(x, ol, oh):
    xb = x[...]
    ol[...] = xb[:, :DH]
    oh[...] = xb[:, DH:]


def _split(x):
    out = jax.ShapeDtypeStruct((NN, DH), jnp.float32)
    return pl.pallas_call(
        _split_body,
        grid=(NN // BN,),
        in_specs=[pl.BlockSpec((BN, D), lambda i: (i, 0))],
        out_specs=[pl.BlockSpec((BN, DH), lambda i: (i, 0))] * 2,
        out_shape=[out, out],
    )(x)


def _sc_body(xl_hbm, xh_hbm, e_hbm, src_hbm, dst_hbm,
             ol_hbm, oh_hbm,
             srcm, dv0, dv1, dv2, dv3, ev0, ev1, xv0, xv1, xv2, xv3,
             zbuf, agg_sh, l0, l1, l2, l3, s0, s1, s2, s3):
    c = lax.axis_index("c")
    s = lax.axis_index("s")
    dst_v = (dv0, dv1, dv2, dv3)
    ev = (ev0, ev1)
    xv = (xv0, xv1, xv2, xv3)
    lsem = (l0, l1, l2, l3)
    ssem = (s0, s1, s2, s3)

    def issue_loads(j, b):
        base = s * EPT + j * CH
        pltpu.async_copy(dst_hbm.at[pl.ds(base, CH)], dst_v[b], lsem[b])
        pltpu.async_copy(e_hbm.at[pl.ds(base, CH)], ev[b % 2], lsem[b])
        idx = srcm.at[pl.ds(j * CH, CH)]

        @pl.when(c == 0)
        def _():
            pltpu.async_copy(xl_hbm.at[idx], xv[b], lsem[b])

        @pl.when(c == 1)
        def _():
            pltpu.async_copy(xh_hbm.at[idx], xv[b], lsem[b])

    def drain_loads(j, b):
        base = s * EPT + j * CH
        pltpu.make_async_copy(dst_hbm.at[pl.ds(base, CH)], dst_v[b],
                              lsem[b]).wait()
        pltpu.make_async_copy(e_hbm.at[pl.ds(base, CH)], ev[b % 2],
                              lsem[b]).wait()
        pltpu.make_async_copy(xl_hbm.at[srcm.at[pl.ds(j * CH, CH)]], xv[b],
                              lsem[b]).wait()

    def compute_half(b, cb):
        hi_mask = jnp.full((LANES,), 0xFFFF0000, jnp.uint32)

        @plsc.parallel_loop(0, CH, unroll=2)
        def row(r):
            for k in range(DH // 32):
                ew = ev[b % 2][r, pl.ds(cb + LANES * k, LANES)]
                e0 = lax.bitcast_convert_type(ew << 16, jnp.float32)
                e1 = lax.bitcast_convert_type(ew & hi_mask, jnp.float32)
                for h, ef in ((0, e0), (1, e1)):
                    sl = pl.ds(32 * k + LANES * h, LANES)
                    xv[b][r, sl] = jnp.maximum(xv[b][r, sl] + ef, 0.0)

    def compute(b):
        @pl.when(c == 0)
        def _():
            compute_half(b, 0)

        @pl.when(c == 1)
        def _():
            compute_half(b, DH // 2)

    def wait_scatter(b):
        pltpu.make_async_copy(xv[b], agg_sh.at[dst_v[b]], ssem[b]).wait()

    def slot(j, u, do_issue, guard_wait):
        drain_loads(j, u)
        compute(u)
        pltpu.async_copy(xv[u], agg_sh.at[dst_v[u]], ssem[u], add=True)
        if guard_wait:
            @pl.when(j >= 2)
            def _():
                wait_scatter((u + 2) % 4)
        else:
            wait_scatter((u + 2) % 4)
        if do_issue:
            issue_loads(j + 2, (u + 2) % 4)

    pltpu.sync_copy(src_hbm.at[pl.ds(s * EPT, EPT)], srcm)
    issue_loads(0, 0)
    issue_loads(1, 1)

    def zero_row(r, carry):
        for k in range(DH // LANES):
            zbuf[r, pl.ds(k * LANES, LANES)] = jnp.zeros((LANES,), jnp.float32)
        return carry
    lax.fori_loop(0, ZR, zero_row, 0)
    for q in range(RPT // ZR):
        pltpu.sync_copy(zbuf, agg_sh.at[pl.ds(s * RPT + q * ZR, ZR)])

    @pl.when(s == NS - 1)
    def _():
        pltpu.sync_copy(zbuf.at[pl.ds(0, NTAIL)],
                        agg_sh.at[pl.ds(NS * RPT, NTAIL)])
    plsc.subcore_barrier()

    def quad(g, carry):
        j = 4 * g
        for u in range(4):
            slot(j + u, u, do_issue=True, guard_wait=(u < 2))
        return carry
    lax.fori_loop(0, (NCH - 2) // 4, quad, 0)
    slot(NCH - 2, 0, do_issue=False, guard_wait=False)
    slot(NCH - 1, 1, do_issue=False, guard_wait=False)
    wait_scatter(0)
    wait_scatter(1)
    plsc.subcore_barrier()

    for q in range(3):
        rows = pl.ds(s * RPT + q * (RPT // 3), RPT // 3)

        @pl.when(c == 0)
        def _():
            pltpu.sync_copy(agg_sh.at[rows], ol_hbm.at[rows])

        @pl.when(c == 1)
        def _():
            pltpu.sync_copy(agg_sh.at[rows], oh_hbm.at[rows])

    @pl.when(s == NS - 1)
    def _():
        tail = pl.ds(NS * RPT, NTAIL)

        @pl.when(c == 0)
        def _():
            pltpu.sync_copy(agg_sh.at[tail], ol_hbm.at[tail])

        @pl.when(c == 1)
        def _():
            pltpu.sync_copy(agg_sh.at[tail], oh_hbm.at[tail])


def _sc_agg(xl, xh, ep, src, dst):
    out = jax.ShapeDtypeStruct((NN, DH), jnp.float32)
    kern = pl.kernel(
        _sc_body,
        out_type=(out, out),
        mesh=plsc.VectorSubcoreMesh(core_axis_name="c", subcore_axis_name="s"),
        scratch_types=(
            [pltpu.VMEM((EPT,), jnp.int32)]
            + [pltpu.VMEM((CH,), jnp.int32)] * 4
            + [pltpu.VMEM((CH, D // 2), jnp.uint32)] * 2
            + [pltpu.VMEM((CH, DH), jnp.float32)] * 4
            + [pltpu.VMEM((ZR, DH), jnp.float32),
               pltpu.VMEM_SHARED((NN, DH), jnp.float32)]
            + [pltpu.SemaphoreType.DMA] * 8
        ),
    )
    return kern(xl, xh, ep, src, dst)


def _mlp1_body(x, al, ah, wa, wb, wp, ba, bb, bp, gs, bt, oh, ohl, ohh):
    xb = x[...]
    hin = xb + jnp.concatenate([al[...], ah[...]], axis=1)
    t = jnp.maximum(jnp.dot(hin, wa[...], preferred_element_type=jnp.float32)
                    + ba[...], 0.0)
    u = jnp.dot(t, wb[...], preferred_element_type=jnp.float32) + bb[...]
    v = jnp.maximum(u, 0.0) * gs[...] + bt[...]
    h = v + jnp.dot(xb, wp[...], preferred_element_type=jnp.float32) + bp[...]
    oh[...] = h
    ohl[...] = h[:, :DH]
    ohh[...] = h[:, DH:]


def _mlp1(x, al, ah, Wa, ba, Wb, bb, Wp, bp, gs, bt):
    fullw = lambda: pl.BlockSpec((D, D), lambda i: (0, 0))
    fullb = lambda: pl.BlockSpec((1, D), lambda i: (0, 0))
    return pl.pallas_call(
        _mlp1_body,
        grid=(NN // BN,),
        in_specs=[
            pl.BlockSpec((BN, D), lambda i: (i, 0)),
            pl.BlockSpec((BN, DH), lambda i: (i, 0)),
            pl.BlockSpec((BN, DH), lambda i: (i, 0)),
            fullw(), fullw(), fullw(),
            fullb(), fullb(), fullb(), fullb(), fullb(),
        ],
        out_specs=[
            pl.BlockSpec((BN, D), lambda i: (i, 0)),
            pl.BlockSpec((BN, DH), lambda i: (i, 0)),
            pl.BlockSpec((BN, DH), lambda i: (i, 0)),
        ],
        out_shape=[
            jax.ShapeDtypeStruct((NN, D), jnp.float32),
            jax.ShapeDtypeStruct((NN, DH), jnp.float32),
            jax.ShapeDtypeStruct((NN, DH), jnp.float32),
        ],
    )(x, al, ah, Wa, Wb, Wp,
      ba.reshape(1, D), bb.reshape(1, D), bp.reshape(1, D),
      gs.reshape(1, D), bt.reshape(1, D))


def _mlp2_pool_body(h, al, ah, wa, wb, wp, ba, bb, bp, gs, bt, bat, out):
    hb = h[...]
    hin = hb + jnp.concatenate([al[...], ah[...]], axis=1)
    t = jnp.maximum(jnp.dot(hin, wa[...], preferred_element_type=jnp.float32)
                    + ba[...], 0.0)
    u = jnp.dot(t, wb[...], preferred_element_type=jnp.float32) + bb[...]
    v = jnp.maximum(u, 0.0) * gs[...] + bt[...]
    h2 = v + jnp.dot(hb, wp[...], preferred_element_type=jnp.float32) + bp[...]

    ids = bat[0, 0, :]
    onehot = (ids[None, :] ==
              lax.broadcasted_iota(jnp.int32, (G, BN), 0)).astype(jnp.float32)

    @pl.when(pl.program_id(0) == 0)
    def _():
        out[...] = jnp.zeros_like(out)
    out[...] += jnp.dot(onehot, h2, preferred_element_type=jnp.float32)


def _mlp2_pool(h, al, ah, Wa, ba, Wb, bb, Wp, bp, gs, bt, batch):
    fullw = lambda: pl.BlockSpec((D, D), lambda i: (0, 0))
    fullb = lambda: pl.BlockSpec((1, D), lambda i: (0, 0))
    nb = NN // BN
    return pl.pallas_call(
        _mlp2_pool_body,
        grid=(nb,),
        in_specs=[
            pl.BlockSpec((BN, D), lambda i: (i, 0)),
            pl.BlockSpec((BN, DH), lambda i: (i, 0)),
            pl.BlockSpec((BN, DH), lambda i: (i, 0)),
            fullw(), fullw(), fullw(),
            fullb(), fullb(), fullb(), fullb(), fullb(),
            pl.BlockSpec((1, 1, BN), lambda i: (i, 0, 0)),
        ],
        out_specs=pl.BlockSpec((G, D), lambda i: (0, 0)),
        out_shape=jax.ShapeDtypeStruct((G, D), jnp.float32),
    )(h, al, ah, Wa, Wb, Wp,
      ba.reshape(1, D), bb.reshape(1, D), bp.reshape(1, D),
      gs.reshape(1, D), bt.reshape(1, D),
      batch.reshape(nb, 1, BN))


def kernel(x, edge_index, edge_attr, batch,
           W1a, b1a, W1b, b1b, We1, be1,
           W2a, b2a, W2b, b2b, We2, be2,
           Wp1, bp1, Wp2, bp2, g1, beta1, g2, beta2):
    src = edge_index[0]
    dst = edge_index[1]
    bn_scale = 1.0 / jnp.sqrt(jnp.float32(1.0 + 1e-5))
    g1s = g1 * bn_scale
    g2s = g2 * bn_scale

    e1p, e2p = _edge_embed(edge_attr, We1, be1, We2, be2)

    xl, xh = _split(x)
    a1l, a1h = _sc_agg(xl, xh, e1p, src, dst)
    h, hl, hh = _mlp1(x, a1l, a1h, W1a, b1a, W1b, b1b, Wp1, bp1, g1s, beta1)
    a2l, a2h = _sc_agg(hl, hh, e2p, src, dst)
    out = _mlp2_pool(h, a2l, a2h, W2a, b2a, W2b, b2b, Wp2, bp2, g2s, beta2,
                     batch)
    return out
